# baseline (speedup 1.0000x reference)
import functools

import jax
import jax.numpy as jnp
import numpy as np
from jax import lax
from jax.experimental import pallas as pl
from jax.experimental.pallas import tpu as pltpu
from jax.experimental.pallas import tpu_sc as plsc

N = 10000
E = 320000
IN = 128
OUT = 256
HID = 32
BN = 2000
GRID = N // BN
SCALE = 1.0 / 16.0

C = 128
NCHUNK = E // C
NWORK = 32
L = 16


def _edge_a_body(y_hbm, x_hbm, u_hbm, c_hbm, dst_hbm, src_hbm, a_hbm,
                 ex_hbm, den_hbm, sa_hbm,
                 idxd0, idxs0, av0, idxd1, idxs1, av1, yr0, xr0, yr1, xr1,
                 dotb, exb, sab, uv, cv, zv, den_sh, sa_sh,
                 semP0, semP1, semG0, semG1, semW):
    cid = lax.axis_index("c")
    sid = lax.axis_index("s")
    wid = cid * 16 + sid

    bufs = ((idxd0, idxs0, av0, yr0, xr0, semP0, semG0),
            (idxd1, idxs1, av1, yr1, xr1, semP1, semG1))

    pltpu.sync_copy(u_hbm, uv)
    pltpu.sync_copy(c_hbm, cv)

    def _zb(i, _):
        zv[pl.ds(i * L, L)] = jnp.zeros((L,), jnp.float32)
        return 0
    lax.fori_loop(0, N // L, _zb, 0)

    @pl.when(sid == 0)
    def _():
        pltpu.sync_copy(zv, den_sh)
        pltpu.sync_copy(zv, sa_sh)
    plsc.subcore_barrier()

    def _off(s):
        return (wid + s * NWORK) * C

    def _valid(s):
        return (wid + s * NWORK) < NCHUNK

    def _start_smalls(s, b):
        idxd, idxs, av, _, _, semP, _ = b
        pltpu.async_copy(dst_hbm.at[pl.ds(_off(s), C)], idxd, semP)
        pltpu.async_copy(src_hbm.at[pl.ds(_off(s), C)], idxs, semP)
        pltpu.async_copy(a_hbm.at[pl.ds(_off(s), C)], av, semP)

    def _wait_smalls(b):
        idxd, idxs, av, _, _, semP, _ = b
        pltpu.make_async_copy(dst_hbm.at[pl.ds(0, C)], idxd, semP).wait()
        pltpu.make_async_copy(src_hbm.at[pl.ds(0, C)], idxs, semP).wait()
        pltpu.make_async_copy(a_hbm.at[pl.ds(0, C)], av, semP).wait()

    def _start_gathers(b):
        idxd, idxs, _, yr, xr, _, semG = b
        pltpu.async_copy(y_hbm.at[idxd], yr, semG)
        pltpu.async_copy(x_hbm.at[idxs], xr, semG)

    def _wait_gathers(b):
        idxd, idxs, _, yr, xr, _, semG = b
        pltpu.make_async_copy(y_hbm.at[idxd], yr, semG).wait()
        pltpu.make_async_copy(x_hbm.at[idxs], xr, semG).wait()

    lane = lax.iota(jnp.int32, L)
    first = lane == 0
    perms = [jnp.bitwise_and(lane + sh, L - 1) for sh in (8, 4, 2, 1)]

    def _hsum(v):
        for p in perms:
            v = v + v.at[p].get(mode="promise_in_bounds")
        return v

    def _slot(t, p):
        b = bufs[p]
        bq = bufs[1 - p]
        idxd, idxs, av, yr, xr, _, _ = b

        @pl.when(_valid(t + 1))
        def _():
            _wait_smalls(bq)
            _start_gathers(bq)

        @pl.when(_valid(t))
        def _():
            _wait_gathers(b)

        def _group(g, _):
            base = g * L
            res = jnp.zeros((L,), jnp.float32)
            for e in range(L):
                row = base + e
                acc = yr[row, pl.ds(0, L)] * xr[row, pl.ds(0, L)]
                for dd in range(1, 8):
                    acc = acc + (yr[row, pl.ds(dd * L, L)]
                                 * xr[row, pl.ds(dd * L, L)])
                res = jnp.where(lane == e, jnp.sum(acc), res)
            dotb[pl.ds(base, L)] = res
            dst16 = idxd[pl.ds(base, L)]
            u16 = plsc.load_gather(uv, [dst16])
            c16 = plsc.load_gather(cv, [dst16])
            a16 = av[pl.ds(base, L)]
            d16 = dotb[pl.ds(base, L)]
            ex16 = jnp.exp(d16 + a16 * u16 + c16)
            exb[pl.ds(base, L)] = ex16
            sab[pl.ds(base, L)] = ex16 * a16
            return 0
        lax.fori_loop(0, C // L, _group, 0)

        @pl.when(_valid(t))
        def _():
            h1 = pltpu.async_copy(exb, ex_hbm.at[pl.ds(_off(t), C)], semW)
            pltpu.sync_copy(exb, den_sh.at[idxd], add=True)
            pltpu.sync_copy(sab, sa_sh.at[idxd], add=True)
            h1.wait()

        @pl.when(_valid(t + 2))
        def _():
            _start_smalls(t + 2, b)

    _start_smalls(0, bufs[0])
    _wait_smalls(bufs[0])
    _start_gathers(bufs[0])
    _start_smalls(1, bufs[1])

    def _pair(i, _):
        _slot(2 * i, 0)
        _slot(2 * i + 1, 1)
        return 0
    lax.fori_loop(0, 40, _pair, 0)
    plsc.subcore_barrier()

    @pl.when(sid == 0)
    def _():
        pltpu.sync_copy(den_sh, zv)
        pltpu.sync_copy(zv, den_hbm.at[pl.ds(cid * N, N)])
        pltpu.sync_copy(sa_sh, zv)
        pltpu.sync_copy(zv, sa_hbm.at[pl.ds(cid * N, N)])


def _edge_a(y, x, u, c, dstI, srcI, a):
    mesh = plsc.VectorSubcoreMesh(core_axis_name="c", subcore_axis_name="s")
    f = pl.kernel(
        _edge_a_body, mesh=mesh,
        compiler_params=pltpu.CompilerParams(needs_layout_passes=False),
        out_type=[
            jax.ShapeDtypeStruct((E,), jnp.float32),
            jax.ShapeDtypeStruct((2 * N,), jnp.float32),
            jax.ShapeDtypeStruct((2 * N,), jnp.float32),
        ],
        scratch_types=[
            pltpu.VMEM((C,), jnp.int32),
            pltpu.VMEM((C,), jnp.int32),
            pltpu.VMEM((C,), jnp.float32),
            pltpu.VMEM((C,), jnp.int32),
            pltpu.VMEM((C,), jnp.int32),
            pltpu.VMEM((C,), jnp.float32),
            pltpu.VMEM((C, 128), jnp.float32),
            pltpu.VMEM((C, 128), jnp.float32),
            pltpu.VMEM((C, 128), jnp.float32),
            pltpu.VMEM((C, 128), jnp.float32),
            pltpu.VMEM((C,), jnp.float32),
            pltpu.VMEM((C,), jnp.float32),
            pltpu.VMEM((C,), jnp.float32),
            pltpu.VMEM((N,), jnp.float32),
            pltpu.VMEM((N,), jnp.float32),
            pltpu.VMEM((N,), jnp.float32),
            pltpu.VMEM_SHARED((N,), jnp.float32),
            pltpu.VMEM_SHARED((N,), jnp.float32),
            pltpu.SemaphoreType.DMA,
            pltpu.SemaphoreType.DMA,
            pltpu.SemaphoreType.DMA,
            pltpu.SemaphoreType.DMA,
            pltpu.SemaphoreType.DMA,
        ],
    )
    return f(y, x, u, c, dstI, srcI, a)


def _dense_pre_body(x_ref, Wq_ref, bq_ref, Wk_ref, Wv_ref, bv_ref, Wsk_ref,
                    bsk_ref, wcol_ref, ccol_ref,
                    y_ref, u_ref, c_ref, v01_ref, sk01_ref, tot_ref):
    i = pl.program_id(0)
    x = x_ref[...]
    q = jax.lax.dot_general(x, Wq_ref[...], (((1,), (0,)), ((), ())),
                            preferred_element_type=jnp.float32) + bq_ref[...]
    y = jax.lax.dot_general(q, Wk_ref[...], (((1,), (1,)), ((), ())),
                            preferred_element_type=jnp.float32)
    y_ref[...] = y * SCALE
    u_ref[...] = jax.lax.dot_general(q, wcol_ref[...], (((1,), (0,)), ((), ())),
                                     preferred_element_type=jnp.float32)
    c_ref[...] = jax.lax.dot_general(q, ccol_ref[...], (((1,), (0,)), ((), ())),
                                     preferred_element_type=jnp.float32)
    v = jax.lax.dot_general(x, Wv_ref[...], (((1,), (0,)), ((), ())),
                            preferred_element_type=jnp.float32) + bv_ref[...]
    v01_ref[0, :, :] = v[:, :128]
    v01_ref[1, :, :] = v[:, 128:]
    sk = jax.lax.dot_general(x, Wsk_ref[...], (((1,), (0,)), ((), ())),
                             preferred_element_type=jnp.float32) + bsk_ref[...]
    sk01_ref[0, :, :] = sk[:, :128]
    sk01_ref[1, :, :] = sk[:, 128:]

    @pl.when(i == 0)
    def _():
        tot_ref[...] = jnp.zeros_like(tot_ref)
    tot_ref[...] = tot_ref[...] + jnp.sum(x[:, 1]).reshape(1, 1)


def _dense_pre(x, Wq, bq, Wk, Wv, bv, Wsk, bsk, wcol, ccol):
    full = lambda s: pl.BlockSpec(s, lambda i: tuple(0 for _ in s))
    return pl.pallas_call(
        _dense_pre_body,
        grid=(GRID,),
        in_specs=[
            pl.BlockSpec((BN, IN), lambda i: (i, 0)),
            full((IN, OUT)), full((1, OUT)), full((IN, OUT)),
            full((IN, OUT)), full((1, OUT)), full((IN, OUT)), full((1, OUT)),
            full((OUT, 1)), full((OUT, 1)),
        ],
        out_specs=[
            pl.BlockSpec((BN, IN), lambda i: (i, 0)),
            pl.BlockSpec((BN, 1), lambda i: (i, 0)),
            pl.BlockSpec((BN, 1), lambda i: (i, 0)),
            pl.BlockSpec((2, BN, 128), lambda i: (0, i, 0)),
            pl.BlockSpec((2, BN, 128), lambda i: (0, i, 0)),
            pl.BlockSpec((1, 1), lambda i: (0, 0)),
        ],
        out_shape=[
            jax.ShapeDtypeStruct((N, IN), jnp.float32),
            jax.ShapeDtypeStruct((N, 1), jnp.float32),
            jax.ShapeDtypeStruct((N, 1), jnp.float32),
            jax.ShapeDtypeStruct((2, N, 128), jnp.float32),
            jax.ShapeDtypeStruct((2, N, 128), jnp.float32),
            jax.ShapeDtypeStruct((1, 1), jnp.float32),
        ],
    )(x, Wq, bq, Wk, Wv, bv, Wsk, bsk, wcol, ccol)


NSLOT_B = 158


def _edge_b_body(vfl_hbm, dst_hbm, src_hbm, ex_hbm, S_hbm,
                 idxd0, idx20, exv0, idxd1, idx21, exv1, vr0, vr1, zbuf,
                 S_sh, semP0, semP1, semG0, semG1, semS0, semS1):
    cid = lax.axis_index("c")
    sid = lax.axis_index("s")
    cN = cid * N

    bufs = ((idxd0, idx20, exv0, vr0, semP0, semG0, semS0),
            (idxd1, idx21, exv1, vr1, semP1, semG1, semS1))

    def _zb(i, _):
        for j in range(8):
            zbuf[i, pl.ds(j * L, L)] = jnp.zeros((L,), jnp.float32)
        return 0
    lax.fori_loop(0, C, _zb, 0)
    row0 = sid * 624
    for k, sz in enumerate((128, 128, 128, 128, 112)):
        pltpu.sync_copy(zbuf.at[pl.ds(0, sz)], S_sh.at[pl.ds(row0 + k * 128, sz)])

    @pl.when(sid == 15)
    def _():
        pltpu.sync_copy(zbuf.at[pl.ds(0, 16)], S_sh.at[pl.ds(9984, 16)])
    plsc.subcore_barrier()

    def _off(s):
        return (sid + s * 16) * C

    def _valid(s):
        return (sid + s * 16) < NCHUNK

    def _start_smalls(s, b):
        idxd, idx2, exv, _, semP, _, _ = b
        pltpu.async_copy(dst_hbm.at[pl.ds(_off(s), C)], idxd, semP)
        pltpu.async_copy(src_hbm.at[pl.ds(_off(s), C)], idx2, semP)
        pltpu.async_copy(ex_hbm.at[pl.ds(_off(s), C)], exv, semP)

    def _wait_smalls(b):
        idxd, idx2, exv, _, semP, _, _ = b
        pltpu.make_async_copy(dst_hbm.at[pl.ds(0, C)], idxd, semP).wait()
        pltpu.make_async_copy(src_hbm.at[pl.ds(0, C)], idx2, semP).wait()
        pltpu.make_async_copy(ex_hbm.at[pl.ds(0, C)], exv, semP).wait()

    def _shift_and_gather(b):
        _, idx2, _, vr, _, semG, _ = b

        def _sh(g, _):
            bb = g * L
            idx2[pl.ds(bb, L)] = idx2[pl.ds(bb, L)] + cN
            return 0
        lax.fori_loop(0, C // L, _sh, 0)
        pltpu.async_copy(vfl_hbm.at[idx2], vr, semG)

    def _wait_gather(b):
        _, idx2, _, vr, _, semG, _ = b
        pltpu.make_async_copy(vfl_hbm.at[idx2], vr, semG).wait()

    def _slot(t, p):
        b = bufs[p]
        bq = bufs[1 - p]
        idxd, _, exv, vr, _, _, semS = b

        @pl.when(_valid(t + 1))
        def _():
            _wait_smalls(bq)
            _shift_and_gather(bq)

        @pl.when(_valid(t))
        def _():
            _wait_gather(b)

        def _group(g, _):
            base = g * L
            ex16 = exv[pl.ds(base, L)]
            exbs = [ex16.at[jnp.full((L,), e, jnp.int32)].get(
                mode="promise_in_bounds") for e in range(L)]
            for e in range(L):
                row = base + e
                for dd in range(8):
                    sl = pl.ds(dd * L, L)
                    vr[row, sl] = vr[row, sl] * exbs[e]
            return 0
        lax.fori_loop(0, C // L, _group, 0)

        @pl.when(_valid(t))
        def _():
            pltpu.sync_copy(vr, S_sh.at[idxd], add=True)

        @pl.when(_valid(t + 2))
        def _():
            _start_smalls(t + 2, b)

    _start_smalls(0, bufs[0])
    _wait_smalls(bufs[0])
    _shift_and_gather(bufs[0])
    _start_smalls(1, bufs[1])

    def _pair(i, _):
        _slot(2 * i, 0)
        _slot(2 * i + 1, 1)
        return 0
    lax.fori_loop(0, NSLOT_B // 2, _pair, 0)
    plsc.subcore_barrier()

    for k, sz in enumerate((128, 128, 128, 128, 112)):
        pltpu.sync_copy(S_sh.at[pl.ds(row0 + k * 128, sz)], zbuf.at[pl.ds(0, sz)])
        pltpu.sync_copy(zbuf.at[pl.ds(0, sz)],
                        S_hbm.at[pl.ds(cN + row0 + k * 128, sz)])

    @pl.when(sid == 15)
    def _():
        pltpu.sync_copy(S_sh.at[pl.ds(9984, 16)], zbuf.at[pl.ds(0, 16)])
        pltpu.sync_copy(zbuf.at[pl.ds(0, 16)], S_hbm.at[pl.ds(cN + 9984, 16)])


def _edge_b(vfl, dstI, srcI, ex):
    mesh = plsc.VectorSubcoreMesh(core_axis_name="c", subcore_axis_name="s")
    f = pl.kernel(
        _edge_b_body, mesh=mesh,
        compiler_params=pltpu.CompilerParams(needs_layout_passes=False),
        out_type=[jax.ShapeDtypeStruct((2 * N, 128), jnp.float32)],
        scratch_types=[
            pltpu.VMEM((C,), jnp.int32),
            pltpu.VMEM((C,), jnp.int32),
            pltpu.VMEM((C,), jnp.float32),
            pltpu.VMEM((C,), jnp.int32),
            pltpu.VMEM((C,), jnp.int32),
            pltpu.VMEM((C,), jnp.float32),
            pltpu.VMEM((C, 128), jnp.float32),
            pltpu.VMEM((C, 128), jnp.float32),
            pltpu.VMEM((C, 128), jnp.float32),
            pltpu.VMEM_SHARED((N, 128), jnp.float32),
            pltpu.SemaphoreType.DMA,
            pltpu.SemaphoreType.DMA,
            pltpu.SemaphoreType.DMA,
            pltpu.SemaphoreType.DMA,
            pltpu.SemaphoreType.DMA,
            pltpu.SemaphoreType.DMA,
        ],
    )
    return f(vfl, dstI, srcI, ex)[0]


def _leaky(t):
    return jnp.where(t > 0, t, 0.01 * t)


def _ln(t, g, bt):
    m = jnp.mean(t, axis=-1, keepdims=True)
    v = jnp.mean((t - m) * (t - m), axis=-1, keepdims=True)
    return (t - m) * jax.lax.rsqrt(v + 1e-5) * g + bt


def _head_body(S_ref, sk_ref, den_ref, sa_ref, x_ref, tot_ref,
               w0_ref, w1_ref, b0_ref, b1e_ref,
               W1a_ref, W1b_ref, W1x_ref, w1r_ref, b1_ref, g1_ref, bt1_ref,
               W2_ref, b2_ref, g2_ref, bt2_ref, W3_ref, b3_ref, conc_ref):
    den = den_ref[0] + den_ref[1]
    sa = sa_ref[0] + sa_ref[1]
    inv = 1.0 / (den + 1e-16)
    agg0 = (S_ref[0] + sa * w0_ref[...] + den * b0_ref[...]) * inv
    agg1 = (S_ref[1] + sa * w1_ref[...] + den * b1e_ref[...]) * inv
    o0 = jnp.maximum(agg0 + sk_ref[0], 0.0)
    o1 = jnp.maximum(agg1 + sk_ref[1], 0.0)
    mm = lambda a, b: jax.lax.dot_general(a, b, (((1,), (0,)), ((), ())),
                                          preferred_element_type=jnp.float32)
    h = mm(o0, W1a_ref[...]) + mm(o1, W1b_ref[...]) + mm(x_ref[...], W1x_ref[...])
    h = h + tot_ref[0, 0] * w1r_ref[...] + b1_ref[...]
    h = _leaky(_ln(h, g1_ref[...], bt1_ref[...]))
    h = _leaky(_ln(mm(h, W2_ref[...]) + b2_ref[...], g2_ref[...], bt2_ref[...]))
    z = mm(h, W3_ref[...]) + b3_ref[...]
    conc_ref[...] = jnp.maximum(z, 0.0) + jnp.log1p(jnp.exp(-jnp.abs(z)))


def _head(S01, sk01, den3, sa3, x, tot, w0, w1, b0, b1e,
          W1a, W1b, W1x, w1r, b1, g1, bt1, W2, b2, g2, bt2, W3, b3):
    full = lambda s: pl.BlockSpec(s, lambda i: tuple(0 for _ in s))
    return pl.pallas_call(
        _head_body,
        grid=(GRID,),
        in_specs=[
            pl.BlockSpec((2, BN, 128), lambda i: (0, i, 0)),
            pl.BlockSpec((2, BN, 128), lambda i: (0, i, 0)),
            pl.BlockSpec((2, BN, 1), lambda i: (0, i, 0)),
            pl.BlockSpec((2, BN, 1), lambda i: (0, i, 0)),
            pl.BlockSpec((BN, IN), lambda i: (i, 0)),
            full((1, 1)),
            full((1, 128)), full((1, 128)), full((1, 128)), full((1, 128)),
            full((128, HID)), full((128, HID)), full((IN, HID)),
            full((1, HID)), full((1, HID)), full((1, HID)), full((1, HID)),
            full((HID, HID)), full((1, HID)), full((1, HID)), full((1, HID)),
            full((HID, 1)), full((1, 1)),
        ],
        out_specs=pl.BlockSpec((BN, 1), lambda i: (i, 0)),
        out_shape=jax.ShapeDtypeStruct((N, 1), jnp.float32),
    )(S01, sk01, den3, sa3, x, tot, w0, w1, b0, b1e,
      W1a, W1b, W1x, w1r, b1, g1, bt1, W2, b2, g2, bt2, W3, b3)


def _norm_body(conc_ref, out_ref):
    cv = conc_ref[...]
    out_ref[...] = cv / (jnp.sum(cv) + 1e-20)


def _normalize(conc_row):
    return pl.pallas_call(
        _norm_body,
        out_shape=jax.ShapeDtypeStruct((1, N), jnp.float32),
    )(conc_row)


def kernel(state, edge_index, edge_attr, pos_feat, Wq, bq, Wk, bk, Wv, bv,
           We, be, Wskip, bskip, W1, b1, g1, bt1, W2, b2, g2, bt2, W3, b3):
    x = jnp.concatenate([state, pos_feat], axis=-1)
    wcol = (We[0] * SCALE).reshape(OUT, 1)
    ccol = ((be + bk) * SCALE).reshape(OUT, 1)
    y, u, c, v01, sk01, tot = _dense_pre(
        x, Wq, bq.reshape(1, OUT), Wk, Wv, bv.reshape(1, OUT),
        Wskip, bskip.reshape(1, OUT), wcol, ccol)

    src = edge_index[0]
    dst = edge_index[1]
    a = edge_attr[:, 0]
    ex, den_fl, sa_fl = _edge_a(y, x, u.reshape(N), c.reshape(N),
                                dst, src, a)
    den3 = den_fl.reshape(2, N, 1)
    sa3 = sa_fl.reshape(2, N, 1)
    Sfl = _edge_b(v01.reshape(2 * N, 128), dst, src, ex)
    S01 = Sfl.reshape(2, N, 128)

    conc = _head(
        S01, sk01, den3, sa3, x, tot,
        (We[0, :128]).reshape(1, 128), (We[0, 128:]).reshape(1, 128),
        (be[:128]).reshape(1, 128), (be[128:]).reshape(1, 128),
        W1[0:128], W1[128:256], W1[257:385], W1[256].reshape(1, HID),
        b1.reshape(1, HID), g1.reshape(1, HID), bt1.reshape(1, HID),
        W2, b2.reshape(1, HID), g2.reshape(1, HID), bt2.reshape(1, HID),
        W3, b3.reshape(1, 1))
    action = _normalize(conc.reshape(1, N))
    return action


# speedup vs baseline: 1.0967x; 1.0967x over previous
import functools

import jax
import jax.numpy as jnp
import numpy as np
from jax import lax
from jax.experimental import pallas as pl
from jax.experimental.pallas import tpu as pltpu
from jax.experimental.pallas import tpu_sc as plsc

N = 10000
E = 320000
IN = 128
OUT = 256
HID = 32
BN = 2000
GRID = N // BN
SCALE = 1.0 / 16.0

C = 128
NCHUNK = E // C
NWORK = 32
L = 16


def _edge_a_body(y_hbm, x_hbm, u_hbm, c_hbm, dst_hbm, src_hbm, a_hbm,
                 ex_hbm, den_hbm, sa_hbm,
                 idxd0, idxs0, av0, idxd1, idxs1, av1, yr0, xr0, yr1, xr1,
                 dotb, exb, sab, uv, cv, zv, den_sh, sa_sh,
                 semP0, semP1, semG0, semG1, semW):
    cid = lax.axis_index("c")
    sid = lax.axis_index("s")
    wid = cid * 16 + sid

    bufs = ((idxd0, idxs0, av0, yr0, xr0, semP0, semG0),
            (idxd1, idxs1, av1, yr1, xr1, semP1, semG1))

    pltpu.sync_copy(u_hbm, uv)
    pltpu.sync_copy(c_hbm, cv)

    def _zb(i, _):
        zv[pl.ds(i * L, L)] = jnp.zeros((L,), jnp.float32)
        return 0
    lax.fori_loop(0, N // L, _zb, 0)

    @pl.when(sid == 0)
    def _():
        pltpu.sync_copy(zv, den_sh)
        pltpu.sync_copy(zv, sa_sh)
    plsc.subcore_barrier()

    def _off(s):
        return (wid + s * NWORK) * C

    def _valid(s):
        return (wid + s * NWORK) < NCHUNK

    def _start_smalls(s, b):
        idxd, idxs, av, _, _, semP, _ = b
        pltpu.async_copy(dst_hbm.at[pl.ds(_off(s), C)], idxd, semP)
        pltpu.async_copy(src_hbm.at[pl.ds(_off(s), C)], idxs, semP)
        pltpu.async_copy(a_hbm.at[pl.ds(_off(s), C)], av, semP)

    def _wait_smalls(b):
        idxd, idxs, av, _, _, semP, _ = b
        pltpu.make_async_copy(dst_hbm.at[pl.ds(0, C)], idxd, semP).wait()
        pltpu.make_async_copy(src_hbm.at[pl.ds(0, C)], idxs, semP).wait()
        pltpu.make_async_copy(a_hbm.at[pl.ds(0, C)], av, semP).wait()

    def _start_gathers(b):
        idxd, idxs, _, yr, xr, _, semG = b
        pltpu.async_copy(y_hbm.at[idxd], yr, semG)
        pltpu.async_copy(x_hbm.at[idxs], xr, semG)

    def _wait_gathers(b):
        idxd, idxs, _, yr, xr, _, semG = b
        pltpu.make_async_copy(y_hbm.at[idxd], yr, semG).wait()
        pltpu.make_async_copy(x_hbm.at[idxs], xr, semG).wait()

    lane = lax.iota(jnp.int32, L)
    first = lane == 0
    perms = [jnp.bitwise_and(lane + sh, L - 1) for sh in (8, 4, 2, 1)]

    def _hsum(v):
        for p in perms:
            v = v + v.at[p].get(mode="promise_in_bounds")
        return v

    def _slot(t, p):
        b = bufs[p]
        bq = bufs[1 - p]
        idxd, idxs, av, yr, xr, _, _ = b

        @pl.when(_valid(t + 1))
        def _():
            _wait_smalls(bq)
            _start_gathers(bq)

        @pl.when(_valid(t))
        def _():
            _wait_gathers(b)

        def _group(g, _):
            base = g * L
            res = jnp.zeros((L,), jnp.float32)
            for e in range(L):
                row = base + e
                acc = yr[row, pl.ds(0, L)] * xr[row, pl.ds(0, L)]
                for dd in range(1, 8):
                    acc = acc + (yr[row, pl.ds(dd * L, L)]
                                 * xr[row, pl.ds(dd * L, L)])
                res = jnp.where(lane == e, jnp.sum(acc), res)
            dotb[pl.ds(base, L)] = res
            dst16 = idxd[pl.ds(base, L)]
            u16 = plsc.load_gather(uv, [dst16])
            c16 = plsc.load_gather(cv, [dst16])
            a16 = av[pl.ds(base, L)]
            d16 = dotb[pl.ds(base, L)]
            ex16 = jnp.exp(d16 + a16 * u16 + c16)
            exb[pl.ds(base, L)] = ex16
            sab[pl.ds(base, L)] = ex16 * a16
            return 0
        lax.fori_loop(0, C // L, _group, 0)

        @pl.when(_valid(t))
        def _():
            h1 = pltpu.async_copy(exb, ex_hbm.at[pl.ds(_off(t), C)], semW)
            pltpu.sync_copy(exb, den_sh.at[idxd], add=True)
            pltpu.sync_copy(sab, sa_sh.at[idxd], add=True)
            h1.wait()

        @pl.when(_valid(t + 2))
        def _():
            _start_smalls(t + 2, b)

    _start_smalls(0, bufs[0])
    _wait_smalls(bufs[0])
    _start_gathers(bufs[0])
    _start_smalls(1, bufs[1])

    def _pair(i, _):
        _slot(2 * i, 0)
        _slot(2 * i + 1, 1)
        return 0
    lax.fori_loop(0, 40, _pair, 0)
    plsc.subcore_barrier()

    @pl.when(sid == 0)
    def _():
        pltpu.sync_copy(den_sh, zv)
        pltpu.sync_copy(zv, den_hbm.at[pl.ds(cid * N, N)])
        pltpu.sync_copy(sa_sh, zv)
        pltpu.sync_copy(zv, sa_hbm.at[pl.ds(cid * N, N)])


def _edge_a(y, x, u, c, dstI, srcI, a):
    mesh = plsc.VectorSubcoreMesh(core_axis_name="c", subcore_axis_name="s")
    f = pl.kernel(
        _edge_a_body, mesh=mesh,
        compiler_params=pltpu.CompilerParams(needs_layout_passes=False),
        out_type=[
            jax.ShapeDtypeStruct((E,), jnp.float32),
            jax.ShapeDtypeStruct((2 * N,), jnp.float32),
            jax.ShapeDtypeStruct((2 * N,), jnp.float32),
        ],
        scratch_types=[
            pltpu.VMEM((C,), jnp.int32),
            pltpu.VMEM((C,), jnp.int32),
            pltpu.VMEM((C,), jnp.float32),
            pltpu.VMEM((C,), jnp.int32),
            pltpu.VMEM((C,), jnp.int32),
            pltpu.VMEM((C,), jnp.float32),
            pltpu.VMEM((C, 128), jnp.float32),
            pltpu.VMEM((C, 128), jnp.float32),
            pltpu.VMEM((C, 128), jnp.float32),
            pltpu.VMEM((C, 128), jnp.float32),
            pltpu.VMEM((C,), jnp.float32),
            pltpu.VMEM((C,), jnp.float32),
            pltpu.VMEM((C,), jnp.float32),
            pltpu.VMEM((N,), jnp.float32),
            pltpu.VMEM((N,), jnp.float32),
            pltpu.VMEM((N,), jnp.float32),
            pltpu.VMEM_SHARED((N,), jnp.float32),
            pltpu.VMEM_SHARED((N,), jnp.float32),
            pltpu.SemaphoreType.DMA,
            pltpu.SemaphoreType.DMA,
            pltpu.SemaphoreType.DMA,
            pltpu.SemaphoreType.DMA,
            pltpu.SemaphoreType.DMA,
        ],
    )
    return f(y, x, u, c, dstI, srcI, a)


def _dense_pre_body(x_ref, Wq_ref, bq_ref, Wk_ref, Wv_ref, bv_ref, Wsk_ref,
                    bsk_ref, wcol_ref, ccol_ref,
                    y_ref, u_ref, c_ref, v01_ref, sk01_ref, tot_ref):
    i = pl.program_id(0)
    x = x_ref[...]
    q = jax.lax.dot_general(x, Wq_ref[...], (((1,), (0,)), ((), ())),
                            preferred_element_type=jnp.float32) + bq_ref[...]
    y = jax.lax.dot_general(q, Wk_ref[...], (((1,), (1,)), ((), ())),
                            preferred_element_type=jnp.float32)
    y_ref[...] = y * SCALE
    u_ref[...] = jax.lax.dot_general(q, wcol_ref[...], (((1,), (0,)), ((), ())),
                                     preferred_element_type=jnp.float32)
    c_ref[...] = jax.lax.dot_general(q, ccol_ref[...], (((1,), (0,)), ((), ())),
                                     preferred_element_type=jnp.float32)
    v = jax.lax.dot_general(x, Wv_ref[...], (((1,), (0,)), ((), ())),
                            preferred_element_type=jnp.float32) + bv_ref[...]
    v01_ref[0, :, :] = v[:, :128]
    v01_ref[1, :, :] = v[:, 128:]
    sk = jax.lax.dot_general(x, Wsk_ref[...], (((1,), (0,)), ((), ())),
                             preferred_element_type=jnp.float32) + bsk_ref[...]
    sk01_ref[0, :, :] = sk[:, :128]
    sk01_ref[1, :, :] = sk[:, 128:]

    @pl.when(i == 0)
    def _():
        tot_ref[...] = jnp.zeros_like(tot_ref)
    tot_ref[...] = tot_ref[...] + jnp.sum(x[:, 1]).reshape(1, 1)


def _dense_pre(x, Wq, bq, Wk, Wv, bv, Wsk, bsk, wcol, ccol):
    full = lambda s: pl.BlockSpec(s, lambda i: tuple(0 for _ in s))
    return pl.pallas_call(
        _dense_pre_body,
        grid=(GRID,),
        in_specs=[
            pl.BlockSpec((BN, IN), lambda i: (i, 0)),
            full((IN, OUT)), full((1, OUT)), full((IN, OUT)),
            full((IN, OUT)), full((1, OUT)), full((IN, OUT)), full((1, OUT)),
            full((OUT, 1)), full((OUT, 1)),
        ],
        out_specs=[
            pl.BlockSpec((BN, IN), lambda i: (i, 0)),
            pl.BlockSpec((BN, 1), lambda i: (i, 0)),
            pl.BlockSpec((BN, 1), lambda i: (i, 0)),
            pl.BlockSpec((2, BN, 128), lambda i: (0, i, 0)),
            pl.BlockSpec((2, BN, 128), lambda i: (0, i, 0)),
            pl.BlockSpec((1, 1), lambda i: (0, 0)),
        ],
        out_shape=[
            jax.ShapeDtypeStruct((N, IN), jnp.float32),
            jax.ShapeDtypeStruct((N, 1), jnp.float32),
            jax.ShapeDtypeStruct((N, 1), jnp.float32),
            jax.ShapeDtypeStruct((2, N, 128), jnp.float32),
            jax.ShapeDtypeStruct((2, N, 128), jnp.float32),
            jax.ShapeDtypeStruct((1, 1), jnp.float32),
        ],
    )(x, Wq, bq, Wk, Wv, bv, Wsk, bsk, wcol, ccol)


NSLOT_B = 158


def _edge_b_body(vfl_hbm, dst_hbm, src_hbm, ex_hbm, S_hbm,
                 idxd0, idx20, exv0, idxd1, idx21, exv1, vr0, vr1, zbuf,
                 S_sh, semP0, semP1, semG0, semG1, semS0, semS1):
    cid = lax.axis_index("c")
    sid = lax.axis_index("s")
    cN = cid * N

    bufs = ((idxd0, idx20, exv0, vr0, semP0, semG0, semS0),
            (idxd1, idx21, exv1, vr1, semP1, semG1, semS1))

    def _zb(i, _):
        for j in range(8):
            zbuf[i, pl.ds(j * L, L)] = jnp.zeros((L,), jnp.float32)
        return 0
    lax.fori_loop(0, C, _zb, 0)
    row0 = sid * 624
    for k, sz in enumerate((128, 128, 128, 128, 112)):
        pltpu.sync_copy(zbuf.at[pl.ds(0, sz)], S_sh.at[pl.ds(row0 + k * 128, sz)])

    @pl.when(sid == 15)
    def _():
        pltpu.sync_copy(zbuf.at[pl.ds(0, 16)], S_sh.at[pl.ds(9984, 16)])
    plsc.subcore_barrier()

    def _off(s):
        return (sid + s * 16) * C

    def _valid(s):
        return (sid + s * 16) < NCHUNK

    def _start_smalls(s, b):
        idxd, idx2, exv, _, semP, _, _ = b
        pltpu.async_copy(dst_hbm.at[pl.ds(_off(s), C)], idxd, semP)
        pltpu.async_copy(src_hbm.at[pl.ds(_off(s), C)], idx2, semP)
        pltpu.async_copy(ex_hbm.at[pl.ds(_off(s), C)], exv, semP)

    def _wait_smalls(b):
        idxd, idx2, exv, _, semP, _, _ = b
        pltpu.make_async_copy(dst_hbm.at[pl.ds(0, C)], idxd, semP).wait()
        pltpu.make_async_copy(src_hbm.at[pl.ds(0, C)], idx2, semP).wait()
        pltpu.make_async_copy(ex_hbm.at[pl.ds(0, C)], exv, semP).wait()

    def _shift_and_gather(b):
        _, idx2, _, vr, _, semG, _ = b

        def _sh(g, _):
            bb = g * L
            idx2[pl.ds(bb, L)] = idx2[pl.ds(bb, L)] + cN
            return 0
        lax.fori_loop(0, C // L, _sh, 0)
        pltpu.async_copy(vfl_hbm.at[idx2], vr, semG)

    def _wait_gather(b):
        _, idx2, _, vr, _, semG, _ = b
        pltpu.make_async_copy(vfl_hbm.at[idx2], vr, semG).wait()

    def _drain_scatter(b):
        idxd, _, _, vr, _, _, semS = b
        pltpu.make_async_copy(vr, S_sh.at[idxd], semS).wait()

    def _slot(t, p):
        b = bufs[p]
        bq = bufs[1 - p]
        idxd, _, exv, vr, _, _, semS = b

        @pl.when((t >= 1) & _valid(t - 1))
        def _():
            _drain_scatter(bq)

        @pl.when(_valid(t + 1))
        def _():
            _wait_smalls(bq)
            _shift_and_gather(bq)

        @pl.when(_valid(t))
        def _():
            _wait_gather(b)

        def _group(g, _):
            base = g * L
            ex16 = exv[pl.ds(base, L)]
            exbs = [ex16.at[jnp.full((L,), e, jnp.int32)].get(
                mode="promise_in_bounds") for e in range(L)]
            for e in range(L):
                row = base + e
                for dd in range(8):
                    sl = pl.ds(dd * L, L)
                    vr[row, sl] = vr[row, sl] * exbs[e]
            return 0
        lax.fori_loop(0, C // L, _group, 0)

        @pl.when(_valid(t))
        def _():
            pltpu.async_copy(vr, S_sh.at[idxd], semS, add=True)

        @pl.when(_valid(t + 2))
        def _():
            _start_smalls(t + 2, b)

    _start_smalls(0, bufs[0])
    _wait_smalls(bufs[0])
    _shift_and_gather(bufs[0])
    _start_smalls(1, bufs[1])

    def _pair(i, _):
        _slot(2 * i, 0)
        _slot(2 * i + 1, 1)
        return 0
    lax.fori_loop(0, NSLOT_B // 2, _pair, 0)
    plsc.subcore_barrier()

    for k, sz in enumerate((128, 128, 128, 128, 112)):
        pltpu.sync_copy(S_sh.at[pl.ds(row0 + k * 128, sz)], zbuf.at[pl.ds(0, sz)])
        pltpu.sync_copy(zbuf.at[pl.ds(0, sz)],
                        S_hbm.at[pl.ds(cN + row0 + k * 128, sz)])

    @pl.when(sid == 15)
    def _():
        pltpu.sync_copy(S_sh.at[pl.ds(9984, 16)], zbuf.at[pl.ds(0, 16)])
        pltpu.sync_copy(zbuf.at[pl.ds(0, 16)], S_hbm.at[pl.ds(cN + 9984, 16)])


def _edge_b(vfl, dstI, srcI, ex):
    mesh = plsc.VectorSubcoreMesh(core_axis_name="c", subcore_axis_name="s")
    f = pl.kernel(
        _edge_b_body, mesh=mesh,
        compiler_params=pltpu.CompilerParams(needs_layout_passes=False),
        out_type=[jax.ShapeDtypeStruct((2 * N, 128), jnp.float32)],
        scratch_types=[
            pltpu.VMEM((C,), jnp.int32),
            pltpu.VMEM((C,), jnp.int32),
            pltpu.VMEM((C,), jnp.float32),
            pltpu.VMEM((C,), jnp.int32),
            pltpu.VMEM((C,), jnp.int32),
            pltpu.VMEM((C,), jnp.float32),
            pltpu.VMEM((C, 128), jnp.float32),
            pltpu.VMEM((C, 128), jnp.float32),
            pltpu.VMEM((C, 128), jnp.float32),
            pltpu.VMEM_SHARED((N, 128), jnp.float32),
            pltpu.SemaphoreType.DMA,
            pltpu.SemaphoreType.DMA,
            pltpu.SemaphoreType.DMA,
            pltpu.SemaphoreType.DMA,
            pltpu.SemaphoreType.DMA,
            pltpu.SemaphoreType.DMA,
        ],
    )
    return f(vfl, dstI, srcI, ex)[0]


def _leaky(t):
    return jnp.where(t > 0, t, 0.01 * t)


def _ln(t, g, bt):
    m = jnp.mean(t, axis=-1, keepdims=True)
    v = jnp.mean((t - m) * (t - m), axis=-1, keepdims=True)
    return (t - m) * jax.lax.rsqrt(v + 1e-5) * g + bt


def _head_body(S_ref, sk_ref, den_ref, sa_ref, x_ref, tot_ref,
               w0_ref, w1_ref, b0_ref, b1e_ref,
               W1a_ref, W1b_ref, W1x_ref, w1r_ref, b1_ref, g1_ref, bt1_ref,
               W2_ref, b2_ref, g2_ref, bt2_ref, W3_ref, b3_ref, conc_ref):
    den = den_ref[0] + den_ref[1]
    sa = sa_ref[0] + sa_ref[1]
    inv = 1.0 / (den + 1e-16)
    agg0 = (S_ref[0] + sa * w0_ref[...] + den * b0_ref[...]) * inv
    agg1 = (S_ref[1] + sa * w1_ref[...] + den * b1e_ref[...]) * inv
    o0 = jnp.maximum(agg0 + sk_ref[0], 0.0)
    o1 = jnp.maximum(agg1 + sk_ref[1], 0.0)
    mm = lambda a, b: jax.lax.dot_general(a, b, (((1,), (0,)), ((), ())),
                                          preferred_element_type=jnp.float32)
    h = mm(o0, W1a_ref[...]) + mm(o1, W1b_ref[...]) + mm(x_ref[...], W1x_ref[...])
    h = h + tot_ref[0, 0] * w1r_ref[...] + b1_ref[...]
    h = _leaky(_ln(h, g1_ref[...], bt1_ref[...]))
    h = _leaky(_ln(mm(h, W2_ref[...]) + b2_ref[...], g2_ref[...], bt2_ref[...]))
    z = mm(h, W3_ref[...]) + b3_ref[...]
    conc_ref[...] = jnp.maximum(z, 0.0) + jnp.log1p(jnp.exp(-jnp.abs(z)))


def _head(S01, sk01, den3, sa3, x, tot, w0, w1, b0, b1e,
          W1a, W1b, W1x, w1r, b1, g1, bt1, W2, b2, g2, bt2, W3, b3):
    full = lambda s: pl.BlockSpec(s, lambda i: tuple(0 for _ in s))
    return pl.pallas_call(
        _head_body,
        grid=(GRID,),
        in_specs=[
            pl.BlockSpec((2, BN, 128), lambda i: (0, i, 0)),
            pl.BlockSpec((2, BN, 128), lambda i: (0, i, 0)),
            pl.BlockSpec((2, BN, 1), lambda i: (0, i, 0)),
            pl.BlockSpec((2, BN, 1), lambda i: (0, i, 0)),
            pl.BlockSpec((BN, IN), lambda i: (i, 0)),
            full((1, 1)),
            full((1, 128)), full((1, 128)), full((1, 128)), full((1, 128)),
            full((128, HID)), full((128, HID)), full((IN, HID)),
            full((1, HID)), full((1, HID)), full((1, HID)), full((1, HID)),
            full((HID, HID)), full((1, HID)), full((1, HID)), full((1, HID)),
            full((HID, 1)), full((1, 1)),
        ],
        out_specs=pl.BlockSpec((BN, 1), lambda i: (i, 0)),
        out_shape=jax.ShapeDtypeStruct((N, 1), jnp.float32),
    )(S01, sk01, den3, sa3, x, tot, w0, w1, b0, b1e,
      W1a, W1b, W1x, w1r, b1, g1, bt1, W2, b2, g2, bt2, W3, b3)


def _norm_body(conc_ref, out_ref):
    cv = conc_ref[...]
    out_ref[...] = cv / (jnp.sum(cv) + 1e-20)


def _normalize(conc_row):
    return pl.pallas_call(
        _norm_body,
        out_shape=jax.ShapeDtypeStruct((1, N), jnp.float32),
    )(conc_row)


def kernel(state, edge_index, edge_attr, pos_feat, Wq, bq, Wk, bk, Wv, bv,
           We, be, Wskip, bskip, W1, b1, g1, bt1, W2, b2, g2, bt2, W3, b3):
    x = jnp.concatenate([state, pos_feat], axis=-1)
    wcol = (We[0] * SCALE).reshape(OUT, 1)
    ccol = ((be + bk) * SCALE).reshape(OUT, 1)
    y, u, c, v01, sk01, tot = _dense_pre(
        x, Wq, bq.reshape(1, OUT), Wk, Wv, bv.reshape(1, OUT),
        Wskip, bskip.reshape(1, OUT), wcol, ccol)

    src = edge_index[0]
    dst = edge_index[1]
    a = edge_attr[:, 0]
    ex, den_fl, sa_fl = _edge_a(y, x, u.reshape(N), c.reshape(N),
                                dst, src, a)
    den3 = den_fl.reshape(2, N, 1)
    sa3 = sa_fl.reshape(2, N, 1)
    Sfl = _edge_b(v01.reshape(2 * N, 128), dst, src, ex)
    S01 = Sfl.reshape(2, N, 128)

    conc = _head(
        S01, sk01, den3, sa3, x, tot,
        (We[0, :128]).reshape(1, 128), (We[0, 128:]).reshape(1, 128),
        (be[:128]).reshape(1, 128), (be[128:]).reshape(1, 128),
        W1[0:128], W1[128:256], W1[257:385], W1[256].reshape(1, HID),
        b1.reshape(1, HID), g1.reshape(1, HID), bt1.reshape(1, HID),
        W2, b2.reshape(1, HID), g2.reshape(1, HID), bt2.reshape(1, HID),
        W3, b3.reshape(1, 1))
    action = _normalize(conc.reshape(1, N))
    return action


# speedup vs baseline: 1.1227x; 1.0237x over previous
import functools

import jax
import jax.numpy as jnp
import numpy as np
from jax import lax
from jax.experimental import pallas as pl
from jax.experimental.pallas import tpu as pltpu
from jax.experimental.pallas import tpu_sc as plsc

N = 10000
E = 320000
IN = 128
OUT = 256
HID = 32
BN = 2000
GRID = N // BN
SCALE = 1.0 / 16.0

C = 128
NCHUNK = E // C
NWORK = 32
L = 16


def _edge_a_body(y_hbm, x_hbm, u_hbm, c_hbm, dst_hbm, src_hbm, a_hbm,
                 ex_hbm, den_hbm, sa_hbm,
                 idxd0, idxs0, av0, idxd1, idxs1, av1, yr0, xr0, yr1, xr1,
                 dotb, exb0, sab0, exb1, sab1, uv, cv, zv, den_sh, sa_sh,
                 semP0, semP1, semG0, semG1, semW0, semW1, semE0, semE1):
    cid = lax.axis_index("c")
    sid = lax.axis_index("s")
    wid = cid * 16 + sid

    bufs = ((idxd0, idxs0, av0, yr0, xr0, semP0, semG0, exb0, sab0, semW0, semE0),
            (idxd1, idxs1, av1, yr1, xr1, semP1, semG1, exb1, sab1, semW1, semE1))

    pltpu.sync_copy(u_hbm, uv)
    pltpu.sync_copy(c_hbm, cv)

    def _zb(i, _):
        zv[pl.ds(i * L, L)] = jnp.zeros((L,), jnp.float32)
        return 0
    lax.fori_loop(0, N // L, _zb, 0)

    @pl.when(sid == 0)
    def _():
        pltpu.sync_copy(zv, den_sh)
        pltpu.sync_copy(zv, sa_sh)
    plsc.subcore_barrier()

    def _off(s):
        return (wid + s * NWORK) * C

    def _valid(s):
        return (wid + s * NWORK) < NCHUNK

    def _drain_outputs(b):
        idxd, _, _, _, _, _, _, exb, sab, semW, semE = b
        pltpu.make_async_copy(exb, ex_hbm.at[pl.ds(0, C)], semE).wait()
        pltpu.make_async_copy(exb, den_sh.at[idxd], semW).wait()
        pltpu.make_async_copy(sab, sa_sh.at[idxd], semW).wait()

    def _start_smalls(s, b):
        idxd, idxs, av, semP = b[0], b[1], b[2], b[5]
        pltpu.async_copy(dst_hbm.at[pl.ds(_off(s), C)], idxd, semP)
        pltpu.async_copy(src_hbm.at[pl.ds(_off(s), C)], idxs, semP)
        pltpu.async_copy(a_hbm.at[pl.ds(_off(s), C)], av, semP)

    def _wait_smalls(b):
        idxd, idxs, av, semP = b[0], b[1], b[2], b[5]
        pltpu.make_async_copy(dst_hbm.at[pl.ds(0, C)], idxd, semP).wait()
        pltpu.make_async_copy(src_hbm.at[pl.ds(0, C)], idxs, semP).wait()
        pltpu.make_async_copy(a_hbm.at[pl.ds(0, C)], av, semP).wait()

    def _start_gathers(b):
        idxd, idxs, yr, xr, semG = b[0], b[1], b[3], b[4], b[6]
        pltpu.async_copy(y_hbm.at[idxd], yr, semG)
        pltpu.async_copy(x_hbm.at[idxs], xr, semG)

    def _wait_gathers(b):
        idxd, idxs, yr, xr, semG = b[0], b[1], b[3], b[4], b[6]
        pltpu.make_async_copy(y_hbm.at[idxd], yr, semG).wait()
        pltpu.make_async_copy(x_hbm.at[idxs], xr, semG).wait()

    lane = lax.iota(jnp.int32, L)
    first = lane == 0
    perms = [jnp.bitwise_and(lane + sh, L - 1) for sh in (8, 4, 2, 1)]

    def _hsum(v):
        for p in perms:
            v = v + v.at[p].get(mode="promise_in_bounds")
        return v

    def _slot(t, p):
        b = bufs[p]
        bq = bufs[1 - p]
        idxd, av, yr, xr = b[0], b[2], b[3], b[4]
        exb, sab, semW, semE = b[7], b[8], b[9], b[10]

        @pl.when(_valid(t + 1))
        def _():
            _wait_smalls(bq)
            _start_gathers(bq)

        @pl.when(_valid(t))
        def _():
            _wait_gathers(b)

        @pl.when((t >= 2) & _valid(t - 2))
        def _():
            _drain_outputs(b)

        def _group(g, _):
            base = g * L
            res = jnp.zeros((L,), jnp.float32)
            for e in range(L):
                row = base + e
                acc = yr[row, pl.ds(0, L)] * xr[row, pl.ds(0, L)]
                for dd in range(1, 8):
                    acc = acc + (yr[row, pl.ds(dd * L, L)]
                                 * xr[row, pl.ds(dd * L, L)])
                res = jnp.where(lane == e, jnp.sum(acc), res)
            dotb[pl.ds(base, L)] = res
            dst16 = idxd[pl.ds(base, L)]
            u16 = plsc.load_gather(uv, [dst16])
            c16 = plsc.load_gather(cv, [dst16])
            a16 = av[pl.ds(base, L)]
            d16 = dotb[pl.ds(base, L)]
            ex16 = jnp.exp(d16 + a16 * u16 + c16)
            exb[pl.ds(base, L)] = ex16
            sab[pl.ds(base, L)] = ex16 * a16
            return 0
        lax.fori_loop(0, C // L, _group, 0)

        @pl.when(_valid(t))
        def _():
            pltpu.async_copy(exb, ex_hbm.at[pl.ds(_off(t), C)], semE)
            pltpu.async_copy(exb, den_sh.at[idxd], semW, add=True)
            pltpu.async_copy(sab, sa_sh.at[idxd], semW, add=True)

        @pl.when(_valid(t + 2))
        def _():
            _start_smalls(t + 2, b)

    _start_smalls(0, bufs[0])
    _wait_smalls(bufs[0])
    _start_gathers(bufs[0])
    _start_smalls(1, bufs[1])

    def _pair(i, _):
        _slot(2 * i, 0)
        _slot(2 * i + 1, 1)
        return 0
    lax.fori_loop(0, 40, _pair, 0)

    @pl.when(_valid(78))
    def _():
        _drain_outputs(bufs[0])

    @pl.when(_valid(79))
    def _():
        _drain_outputs(bufs[1])
    plsc.subcore_barrier()

    @pl.when(sid == 0)
    def _():
        pltpu.sync_copy(den_sh, zv)
        pltpu.sync_copy(zv, den_hbm.at[pl.ds(cid * N, N)])
        pltpu.sync_copy(sa_sh, zv)
        pltpu.sync_copy(zv, sa_hbm.at[pl.ds(cid * N, N)])


def _edge_a(y, x, u, c, dstI, srcI, a):
    mesh = plsc.VectorSubcoreMesh(core_axis_name="c", subcore_axis_name="s")
    f = pl.kernel(
        _edge_a_body, mesh=mesh,
        compiler_params=pltpu.CompilerParams(needs_layout_passes=False),
        out_type=[
            jax.ShapeDtypeStruct((E,), jnp.float32),
            jax.ShapeDtypeStruct((2 * N,), jnp.float32),
            jax.ShapeDtypeStruct((2 * N,), jnp.float32),
        ],
        scratch_types=[
            pltpu.VMEM((C,), jnp.int32),
            pltpu.VMEM((C,), jnp.int32),
            pltpu.VMEM((C,), jnp.float32),
            pltpu.VMEM((C,), jnp.int32),
            pltpu.VMEM((C,), jnp.int32),
            pltpu.VMEM((C,), jnp.float32),
            pltpu.VMEM((C, 128), jnp.float32),
            pltpu.VMEM((C, 128), jnp.float32),
            pltpu.VMEM((C, 128), jnp.float32),
            pltpu.VMEM((C, 128), jnp.float32),
            pltpu.VMEM((C,), jnp.float32),
            pltpu.VMEM((C,), jnp.float32),
            pltpu.VMEM((C,), jnp.float32),
            pltpu.VMEM((C,), jnp.float32),
            pltpu.VMEM((C,), jnp.float32),
            pltpu.VMEM((N,), jnp.float32),
            pltpu.VMEM((N,), jnp.float32),
            pltpu.VMEM((N,), jnp.float32),
            pltpu.VMEM_SHARED((N,), jnp.float32),
            pltpu.VMEM_SHARED((N,), jnp.float32),
            pltpu.SemaphoreType.DMA,
            pltpu.SemaphoreType.DMA,
            pltpu.SemaphoreType.DMA,
            pltpu.SemaphoreType.DMA,
            pltpu.SemaphoreType.DMA,
            pltpu.SemaphoreType.DMA,
            pltpu.SemaphoreType.DMA,
            pltpu.SemaphoreType.DMA,
        ],
    )
    return f(y, x, u, c, dstI, srcI, a)


def _dense_pre_body(x_ref, Wq_ref, bq_ref, Wk_ref, Wv_ref, bv_ref, Wsk_ref,
                    bsk_ref, wcol_ref, ccol_ref,
                    y_ref, u_ref, c_ref, v01_ref, sk01_ref, tot_ref):
    i = pl.program_id(0)
    x = x_ref[...]
    q = jax.lax.dot_general(x, Wq_ref[...], (((1,), (0,)), ((), ())),
                            preferred_element_type=jnp.float32) + bq_ref[...]
    y = jax.lax.dot_general(q, Wk_ref[...], (((1,), (1,)), ((), ())),
                            preferred_element_type=jnp.float32)
    y_ref[...] = y * SCALE
    u_ref[...] = jax.lax.dot_general(q, wcol_ref[...], (((1,), (0,)), ((), ())),
                                     preferred_element_type=jnp.float32)
    c_ref[...] = jax.lax.dot_general(q, ccol_ref[...], (((1,), (0,)), ((), ())),
                                     preferred_element_type=jnp.float32)
    v = jax.lax.dot_general(x, Wv_ref[...], (((1,), (0,)), ((), ())),
                            preferred_element_type=jnp.float32) + bv_ref[...]
    v01_ref[0, :, :] = v[:, :128]
    v01_ref[1, :, :] = v[:, 128:]
    sk = jax.lax.dot_general(x, Wsk_ref[...], (((1,), (0,)), ((), ())),
                             preferred_element_type=jnp.float32) + bsk_ref[...]
    sk01_ref[0, :, :] = sk[:, :128]
    sk01_ref[1, :, :] = sk[:, 128:]

    @pl.when(i == 0)
    def _():
        tot_ref[...] = jnp.zeros_like(tot_ref)
    tot_ref[...] = tot_ref[...] + jnp.sum(x[:, 1]).reshape(1, 1)


def _dense_pre(x, Wq, bq, Wk, Wv, bv, Wsk, bsk, wcol, ccol):
    full = lambda s: pl.BlockSpec(s, lambda i: tuple(0 for _ in s))
    return pl.pallas_call(
        _dense_pre_body,
        grid=(GRID,),
        in_specs=[
            pl.BlockSpec((BN, IN), lambda i: (i, 0)),
            full((IN, OUT)), full((1, OUT)), full((IN, OUT)),
            full((IN, OUT)), full((1, OUT)), full((IN, OUT)), full((1, OUT)),
            full((OUT, 1)), full((OUT, 1)),
        ],
        out_specs=[
            pl.BlockSpec((BN, IN), lambda i: (i, 0)),
            pl.BlockSpec((BN, 1), lambda i: (i, 0)),
            pl.BlockSpec((BN, 1), lambda i: (i, 0)),
            pl.BlockSpec((2, BN, 128), lambda i: (0, i, 0)),
            pl.BlockSpec((2, BN, 128), lambda i: (0, i, 0)),
            pl.BlockSpec((1, 1), lambda i: (0, 0)),
        ],
        out_shape=[
            jax.ShapeDtypeStruct((N, IN), jnp.float32),
            jax.ShapeDtypeStruct((N, 1), jnp.float32),
            jax.ShapeDtypeStruct((N, 1), jnp.float32),
            jax.ShapeDtypeStruct((2, N, 128), jnp.float32),
            jax.ShapeDtypeStruct((2, N, 128), jnp.float32),
            jax.ShapeDtypeStruct((1, 1), jnp.float32),
        ],
    )(x, Wq, bq, Wk, Wv, bv, Wsk, bsk, wcol, ccol)


NSLOT_B = 158


def _edge_b_body(vfl_hbm, dst_hbm, src_hbm, ex_hbm, S_hbm,
                 idxd0, idx20, exv0, idxd1, idx21, exv1, vr0, vr1, zbuf,
                 S_sh, semP0, semP1, semG0, semG1, semS0, semS1):
    cid = lax.axis_index("c")
    sid = lax.axis_index("s")
    cN = cid * N

    bufs = ((idxd0, idx20, exv0, vr0, semP0, semG0, semS0),
            (idxd1, idx21, exv1, vr1, semP1, semG1, semS1))

    def _zb(i, _):
        for j in range(8):
            zbuf[i, pl.ds(j * L, L)] = jnp.zeros((L,), jnp.float32)
        return 0
    lax.fori_loop(0, C, _zb, 0)
    row0 = sid * 624
    for k, sz in enumerate((128, 128, 128, 128, 112)):
        pltpu.sync_copy(zbuf.at[pl.ds(0, sz)], S_sh.at[pl.ds(row0 + k * 128, sz)])

    @pl.when(sid == 15)
    def _():
        pltpu.sync_copy(zbuf.at[pl.ds(0, 16)], S_sh.at[pl.ds(9984, 16)])
    plsc.subcore_barrier()

    def _off(s):
        return (sid + s * 16) * C

    def _valid(s):
        return (sid + s * 16) < NCHUNK

    def _start_smalls(s, b):
        idxd, idx2, exv, _, semP, _, _ = b
        pltpu.async_copy(dst_hbm.at[pl.ds(_off(s), C)], idxd, semP)
        pltpu.async_copy(src_hbm.at[pl.ds(_off(s), C)], idx2, semP)
        pltpu.async_copy(ex_hbm.at[pl.ds(_off(s), C)], exv, semP)

    def _wait_smalls(b):
        idxd, idx2, exv, _, semP, _, _ = b
        pltpu.make_async_copy(dst_hbm.at[pl.ds(0, C)], idxd, semP).wait()
        pltpu.make_async_copy(src_hbm.at[pl.ds(0, C)], idx2, semP).wait()
        pltpu.make_async_copy(ex_hbm.at[pl.ds(0, C)], exv, semP).wait()

    def _shift_and_gather(b):
        _, idx2, _, vr, _, semG, _ = b

        def _sh(g, _):
            bb = g * L
            idx2[pl.ds(bb, L)] = idx2[pl.ds(bb, L)] + cN
            return 0
        lax.fori_loop(0, C // L, _sh, 0)
        pltpu.async_copy(vfl_hbm.at[idx2], vr, semG)

    def _wait_gather(b):
        _, idx2, _, vr, _, semG, _ = b
        pltpu.make_async_copy(vfl_hbm.at[idx2], vr, semG).wait()

    def _drain_scatter(b):
        idxd, _, _, vr, _, _, semS = b
        pltpu.make_async_copy(vr, S_sh.at[idxd], semS).wait()

    def _slot(t, p):
        b = bufs[p]
        bq = bufs[1 - p]
        idxd, _, exv, vr, _, _, semS = b

        @pl.when((t >= 1) & _valid(t - 1))
        def _():
            _drain_scatter(bq)

        @pl.when(_valid(t + 1))
        def _():
            _wait_smalls(bq)
            _shift_and_gather(bq)

        @pl.when(_valid(t))
        def _():
            _wait_gather(b)

        def _group(g, _):
            base = g * L
            ex16 = exv[pl.ds(base, L)]
            exbs = [ex16.at[jnp.full((L,), e, jnp.int32)].get(
                mode="promise_in_bounds") for e in range(L)]
            for e in range(L):
                row = base + e
                for dd in range(8):
                    sl = pl.ds(dd * L, L)
                    vr[row, sl] = vr[row, sl] * exbs[e]
            return 0
        lax.fori_loop(0, C // L, _group, 0)

        @pl.when(_valid(t))
        def _():
            pltpu.async_copy(vr, S_sh.at[idxd], semS, add=True)

        @pl.when(_valid(t + 2))
        def _():
            _start_smalls(t + 2, b)

    _start_smalls(0, bufs[0])
    _wait_smalls(bufs[0])
    _shift_and_gather(bufs[0])
    _start_smalls(1, bufs[1])

    def _pair(i, _):
        _slot(2 * i, 0)
        _slot(2 * i + 1, 1)
        return 0
    lax.fori_loop(0, NSLOT_B // 2, _pair, 0)
    plsc.subcore_barrier()

    for k, sz in enumerate((128, 128, 128, 128, 112)):
        pltpu.sync_copy(S_sh.at[pl.ds(row0 + k * 128, sz)], zbuf.at[pl.ds(0, sz)])
        pltpu.sync_copy(zbuf.at[pl.ds(0, sz)],
                        S_hbm.at[pl.ds(cN + row0 + k * 128, sz)])

    @pl.when(sid == 15)
    def _():
        pltpu.sync_copy(S_sh.at[pl.ds(9984, 16)], zbuf.at[pl.ds(0, 16)])
        pltpu.sync_copy(zbuf.at[pl.ds(0, 16)], S_hbm.at[pl.ds(cN + 9984, 16)])


def _edge_b(vfl, dstI, srcI, ex):
    mesh = plsc.VectorSubcoreMesh(core_axis_name="c", subcore_axis_name="s")
    f = pl.kernel(
        _edge_b_body, mesh=mesh,
        compiler_params=pltpu.CompilerParams(needs_layout_passes=False),
        out_type=[jax.ShapeDtypeStruct((2 * N, 128), jnp.float32)],
        scratch_types=[
            pltpu.VMEM((C,), jnp.int32),
            pltpu.VMEM((C,), jnp.int32),
            pltpu.VMEM((C,), jnp.float32),
            pltpu.VMEM((C,), jnp.int32),
            pltpu.VMEM((C,), jnp.int32),
            pltpu.VMEM((C,), jnp.float32),
            pltpu.VMEM((C, 128), jnp.float32),
            pltpu.VMEM((C, 128), jnp.float32),
            pltpu.VMEM((C, 128), jnp.float32),
            pltpu.VMEM_SHARED((N, 128), jnp.float32),
            pltpu.SemaphoreType.DMA,
            pltpu.SemaphoreType.DMA,
            pltpu.SemaphoreType.DMA,
            pltpu.SemaphoreType.DMA,
            pltpu.SemaphoreType.DMA,
            pltpu.SemaphoreType.DMA,
        ],
    )
    return f(vfl, dstI, srcI, ex)[0]


def _leaky(t):
    return jnp.where(t > 0, t, 0.01 * t)


def _ln(t, g, bt):
    m = jnp.mean(t, axis=-1, keepdims=True)
    v = jnp.mean((t - m) * (t - m), axis=-1, keepdims=True)
    return (t - m) * jax.lax.rsqrt(v + 1e-5) * g + bt


def _head_body(S_ref, sk_ref, den_ref, sa_ref, x_ref, tot_ref,
               w0_ref, w1_ref, b0_ref, b1e_ref,
               W1a_ref, W1b_ref, W1x_ref, w1r_ref, b1_ref, g1_ref, bt1_ref,
               W2_ref, b2_ref, g2_ref, bt2_ref, W3_ref, b3_ref, conc_ref):
    den = den_ref[0] + den_ref[1]
    sa = sa_ref[0] + sa_ref[1]
    inv = 1.0 / (den + 1e-16)
    agg0 = (S_ref[0] + sa * w0_ref[...] + den * b0_ref[...]) * inv
    agg1 = (S_ref[1] + sa * w1_ref[...] + den * b1e_ref[...]) * inv
    o0 = jnp.maximum(agg0 + sk_ref[0], 0.0)
    o1 = jnp.maximum(agg1 + sk_ref[1], 0.0)
    mm = lambda a, b: jax.lax.dot_general(a, b, (((1,), (0,)), ((), ())),
                                          preferred_element_type=jnp.float32)
    h = mm(o0, W1a_ref[...]) + mm(o1, W1b_ref[...]) + mm(x_ref[...], W1x_ref[...])
    h = h + tot_ref[0, 0] * w1r_ref[...] + b1_ref[...]
    h = _leaky(_ln(h, g1_ref[...], bt1_ref[...]))
    h = _leaky(_ln(mm(h, W2_ref[...]) + b2_ref[...], g2_ref[...], bt2_ref[...]))
    z = mm(h, W3_ref[...]) + b3_ref[...]
    conc_ref[...] = jnp.maximum(z, 0.0) + jnp.log1p(jnp.exp(-jnp.abs(z)))


def _head(S01, sk01, den3, sa3, x, tot, w0, w1, b0, b1e,
          W1a, W1b, W1x, w1r, b1, g1, bt1, W2, b2, g2, bt2, W3, b3):
    full = lambda s: pl.BlockSpec(s, lambda i: tuple(0 for _ in s))
    return pl.pallas_call(
        _head_body,
        grid=(GRID,),
        in_specs=[
            pl.BlockSpec((2, BN, 128), lambda i: (0, i, 0)),
            pl.BlockSpec((2, BN, 128), lambda i: (0, i, 0)),
            pl.BlockSpec((2, BN, 1), lambda i: (0, i, 0)),
            pl.BlockSpec((2, BN, 1), lambda i: (0, i, 0)),
            pl.BlockSpec((BN, IN), lambda i: (i, 0)),
            full((1, 1)),
            full((1, 128)), full((1, 128)), full((1, 128)), full((1, 128)),
            full((128, HID)), full((128, HID)), full((IN, HID)),
            full((1, HID)), full((1, HID)), full((1, HID)), full((1, HID)),
            full((HID, HID)), full((1, HID)), full((1, HID)), full((1, HID)),
            full((HID, 1)), full((1, 1)),
        ],
        out_specs=pl.BlockSpec((BN, 1), lambda i: (i, 0)),
        out_shape=jax.ShapeDtypeStruct((N, 1), jnp.float32),
    )(S01, sk01, den3, sa3, x, tot, w0, w1, b0, b1e,
      W1a, W1b, W1x, w1r, b1, g1, bt1, W2, b2, g2, bt2, W3, b3)


def _norm_body(conc_ref, out_ref):
    cv = conc_ref[...]
    out_ref[...] = cv / (jnp.sum(cv) + 1e-20)


def _normalize(conc_row):
    return pl.pallas_call(
        _norm_body,
        out_shape=jax.ShapeDtypeStruct((1, N), jnp.float32),
    )(conc_row)


def kernel(state, edge_index, edge_attr, pos_feat, Wq, bq, Wk, bk, Wv, bv,
           We, be, Wskip, bskip, W1, b1, g1, bt1, W2, b2, g2, bt2, W3, b3):
    x = jnp.concatenate([state, pos_feat], axis=-1)
    wcol = (We[0] * SCALE).reshape(OUT, 1)
    ccol = ((be + bk) * SCALE).reshape(OUT, 1)
    y, u, c, v01, sk01, tot = _dense_pre(
        x, Wq, bq.reshape(1, OUT), Wk, Wv, bv.reshape(1, OUT),
        Wskip, bskip.reshape(1, OUT), wcol, ccol)

    src = edge_index[0]
    dst = edge_index[1]
    a = edge_attr[:, 0]
    ex, den_fl, sa_fl = _edge_a(y, x, u.reshape(N), c.reshape(N),
                                dst, src, a)
    den3 = den_fl.reshape(2, N, 1)
    sa3 = sa_fl.reshape(2, N, 1)
    Sfl = _edge_b(v01.reshape(2 * N, 128), dst, src, ex)
    S01 = Sfl.reshape(2, N, 128)

    conc = _head(
        S01, sk01, den3, sa3, x, tot,
        (We[0, :128]).reshape(1, 128), (We[0, 128:]).reshape(1, 128),
        (be[:128]).reshape(1, 128), (be[128:]).reshape(1, 128),
        W1[0:128], W1[128:256], W1[257:385], W1[256].reshape(1, HID),
        b1.reshape(1, HID), g1.reshape(1, HID), bt1.reshape(1, HID),
        W2, b2.reshape(1, HID), g2.reshape(1, HID), bt2.reshape(1, HID),
        W3, b3.reshape(1, 1))
    action = _normalize(conc.reshape(1, N))
    return action
